# baseline (speedup 1.0000x reference)
import functools

import jax
import jax.numpy as jnp
from jax import lax
from jax.experimental import pallas as pl
from jax.experimental.pallas import tpu as pltpu
from jax.experimental.pallas import tpu_sc as plsc

N_NODES = 10000
D = 128
E = 320000
NC = 2
NS = 16
NW = NC * NS
K = 80
EPW = E // NW
CPW = EPW // K
TILE_ROWS = N_NODES // NS
ZR = 125


def _sc_body(x_hbm, src_hbm, dst_hbm, ew_hbm, out_hbm,
             rows_v, src_v, dst_v, ew_v, zero_v, agg_sh, g0, g1, m0, m1):
    c = lax.axis_index("c")
    s = lax.axis_index("s")
    wid = s * NC + c

    def zb(i, carry):
        for j in range(4):
            zero_v[i, pl.ds(32 * j, 32)] = jnp.zeros((32,), jnp.bfloat16)
        return carry
    lax.fori_loop(0, ZR, zb, 0)
    base = s * TILE_ROWS
    for k in range(TILE_ROWS // ZR):
        pltpu.sync_copy(zero_v, agg_sh.at[pl.ds(base + k * ZR, ZR)])

    wrow = wid * CPW

    def meta_fetch(b, ci, sem):
        pltpu.async_copy(src_hbm.at[wrow + ci], src_v.at[b], sem)
        pltpu.async_copy(dst_hbm.at[wrow + ci], dst_v.at[b], sem)
        pltpu.async_copy(ew_hbm.at[wrow + ci], ew_v.at[b], sem)

    def meta_wait(b, ci, sem):
        pltpu.make_async_copy(src_hbm.at[wrow + ci], src_v.at[b], sem).wait()
        pltpu.make_async_copy(dst_hbm.at[wrow + ci], dst_v.at[b], sem).wait()
        pltpu.make_async_copy(ew_hbm.at[wrow + ci], ew_v.at[b], sem).wait()

    def gather(b, ci, gsem, msem):
        meta_wait(b, ci, msem)
        pltpu.async_copy(x_hbm.at[src_v.at[b]], rows_v.at[b], gsem)

    def process(b, ci, gsem, msem):
        pltpu.make_async_copy(
            x_hbm.at[src_v.at[b]], rows_v.at[b], gsem).wait()

        def rowfn(g, rc):
            wv = ew_v[b, pl.ds(16 * g, 16)]
            for l in range(16):
                ws = jnp.full((16,), wv[l], jnp.int32)
                w = plsc.bitcast(ws, jnp.bfloat16)
                r = 16 * g + l
                for j in range(4):
                    sl = pl.ds(32 * j, 32)
                    rows_v[b, r, sl] = rows_v[b, r, sl] * w
            return rc
        lax.fori_loop(0, K // 16, rowfn, 0)
        pltpu.sync_copy(rows_v.at[b], agg_sh.at[dst_v.at[b]], add=True)
        ci_next = jnp.minimum(ci + 2, CPW - 1)
        meta_fetch(b, ci_next, msem)

    plsc.subcore_barrier()
    meta_fetch(0, 0, m0)
    meta_fetch(1, 1, m1)
    gather(0, 0, g0, m0)

    def pair(t, carry):
        ca = 2 * t
        gather(1, ca + 1, g1, m1)
        process(0, ca, g0, m0)
        gather(0, ca + 2, g0, m0)
        process(1, ca + 1, g1, m1)
        return carry
    lax.fori_loop(0, (CPW - 1) // 2, pair, 0)
    process(0, CPW - 1, g0, m0)
    meta_wait(0, CPW - 1, m0)
    meta_wait(1, CPW - 1, m1)

    plsc.subcore_barrier()
    for k in range(TILE_ROWS // ZR):
        pltpu.sync_copy(agg_sh.at[pl.ds(base + k * ZR, ZR)],
                        out_hbm.at[c, pl.ds(base + k * ZR, ZR)])


@functools.partial(
    pl.kernel,
    out_type=jax.ShapeDtypeStruct((NC, N_NODES, D), jnp.bfloat16),
    mesh=plsc.VectorSubcoreMesh(core_axis_name="c", subcore_axis_name="s"),
    compiler_params=pltpu.CompilerParams(use_tc_tiling_on_sc=False,
                                         needs_layout_passes=False),
    scratch_types=[
        pltpu.VMEM((2, K, D), jnp.bfloat16),
        pltpu.VMEM((2, K), jnp.int32),
        pltpu.VMEM((2, K), jnp.int32),
        pltpu.VMEM((2, K), jnp.int32),
        pltpu.VMEM((ZR, D), jnp.bfloat16),
        pltpu.VMEM_SHARED((N_NODES, D), jnp.bfloat16),
        pltpu.SemaphoreType.DMA,
        pltpu.SemaphoreType.DMA,
        pltpu.SemaphoreType.DMA,
        pltpu.SemaphoreType.DMA,
    ],
)
def _sc_agg(x_hbm, src_hbm, dst_hbm, ew_hbm, out_hbm, *scratch):
    _sc_body(x_hbm, src_hbm, dst_hbm, ew_hbm, out_hbm, *scratch)


BM = 400


def _tc_body(p_ref, w_ref, b_ref, o_ref):
    acc = (p_ref[0].astype(jnp.float32) + p_ref[1].astype(jnp.float32))
    o_ref[...] = (
        jnp.dot(acc, w_ref[...], preferred_element_type=jnp.float32)
        + b_ref[...]
    )


def _tc_finish(partials, W2, b2):
    return pl.pallas_call(
        _tc_body,
        grid=(N_NODES // BM,),
        in_specs=[
            pl.BlockSpec((NC, BM, D), lambda i: (0, i, 0)),
            pl.BlockSpec((D, D), lambda i: (0, 0)),
            pl.BlockSpec((1, D), lambda i: (0, 0)),
        ],
        out_specs=pl.BlockSpec((BM, D), lambda i: (i, 0)),
        out_shape=jax.ShapeDtypeStruct((N_NODES, D), jnp.float32),
    )(partials, W2, b2.reshape(1, D))


def kernel(x, edge_index, edge_weight, W1, b1, W2, b2):
    src = edge_index[0].astype(jnp.int32).reshape(E // K, K)
    dst = edge_index[1].astype(jnp.int32).reshape(E // K, K)
    wb = lax.bitcast_convert_type(
        edge_weight.astype(jnp.bfloat16).reshape(E // K, K),
        jnp.uint16).astype(jnp.uint32)
    ew = ((wb << 16) | wb).astype(jnp.int32)
    partials = _sc_agg(x.astype(jnp.bfloat16), src, dst, ew)
    return _tc_finish(partials, W2, b2)


# speedup vs baseline: 1.4997x; 1.4997x over previous
import functools

import jax
import jax.numpy as jnp
from jax import lax
from jax.experimental import pallas as pl
from jax.experimental.pallas import tpu as pltpu
from jax.experimental.pallas import tpu_sc as plsc

N_NODES = 10000
D = 128
E = 320000
NC = 2
NS = 16
NW = NC * NS
K = 80
EPW = E // NW
CPW = EPW // K
TILE_ROWS = N_NODES // NS
ZR = 125


def _sc_body(x_hbm, src_hbm, dst_hbm, ew_hbm, out_hbm,
             rows_v, src_v, dst_v, ew_v, zero_v, agg_sh, g0, g1, m0, m1):
    c = lax.axis_index("c")
    s = lax.axis_index("s")
    wid = s * NC + c

    def zb(i, carry):
        for j in range(8):
            zero_v[i, pl.ds(16 * j, 16)] = jnp.zeros((16,), jnp.float32)
        return carry
    lax.fori_loop(0, ZR, zb, 0)
    base = s * TILE_ROWS
    for k in range(TILE_ROWS // ZR):
        pltpu.sync_copy(zero_v, agg_sh.at[pl.ds(base + k * ZR, ZR)])

    wrow = wid * CPW

    def meta_fetch(b, ci, sem):
        pltpu.async_copy(src_hbm.at[wrow + ci], src_v.at[b], sem)
        pltpu.async_copy(dst_hbm.at[wrow + ci], dst_v.at[b], sem)
        pltpu.async_copy(ew_hbm.at[wrow + ci], ew_v.at[b], sem)

    def meta_wait(b, ci, sem):
        pltpu.make_async_copy(src_hbm.at[wrow + ci], src_v.at[b], sem).wait()
        pltpu.make_async_copy(dst_hbm.at[wrow + ci], dst_v.at[b], sem).wait()
        pltpu.make_async_copy(ew_hbm.at[wrow + ci], ew_v.at[b], sem).wait()

    def gather(b, ci, gsem, msem):
        meta_wait(b, ci, msem)
        pltpu.async_copy(x_hbm.at[src_v.at[b]], rows_v.at[b], gsem)

    def process(b, ci, gsem, msem):
        pltpu.make_async_copy(
            x_hbm.at[src_v.at[b]], rows_v.at[b], gsem).wait()

        @plsc.parallel_loop(0, K // 16)
        def rowfn(g):
            wv = ew_v[b, pl.ds(16 * g, 16)]
            ws = [wv[l] for l in range(16)]
            for l in range(16):
                r = 16 * g + l
                for j in range(8):
                    sl = pl.ds(16 * j, 16)
                    rows_v[b, r, sl] = rows_v[b, r, sl] * ws[l]
        pltpu.sync_copy(rows_v.at[b], agg_sh.at[dst_v.at[b]], add=True)
        ci_next = jnp.minimum(ci + 2, CPW - 1)
        meta_fetch(b, ci_next, msem)

    plsc.subcore_barrier()
    meta_fetch(0, 0, m0)
    meta_fetch(1, 1, m1)
    gather(0, 0, g0, m0)

    def pair(t, carry):
        ca = 2 * t
        gather(1, ca + 1, g1, m1)
        process(0, ca, g0, m0)
        gather(0, ca + 2, g0, m0)
        process(1, ca + 1, g1, m1)
        return carry
    lax.fori_loop(0, (CPW - 1) // 2, pair, 0)
    process(0, CPW - 1, g0, m0)
    meta_wait(0, CPW - 1, m0)
    meta_wait(1, CPW - 1, m1)

    plsc.subcore_barrier()
    for k in range(TILE_ROWS // ZR):
        pltpu.sync_copy(agg_sh.at[pl.ds(base + k * ZR, ZR)],
                        out_hbm.at[c, pl.ds(base + k * ZR, ZR)])


@functools.partial(
    pl.kernel,
    out_type=jax.ShapeDtypeStruct((NC, N_NODES, D), jnp.float32),
    mesh=plsc.VectorSubcoreMesh(core_axis_name="c", subcore_axis_name="s"),
    compiler_params=pltpu.CompilerParams(use_tc_tiling_on_sc=False,
                                         needs_layout_passes=False),
    scratch_types=[
        pltpu.VMEM((2, K, D), jnp.float32),
        pltpu.VMEM((2, K), jnp.int32),
        pltpu.VMEM((2, K), jnp.int32),
        pltpu.VMEM((2, K), jnp.float32),
        pltpu.VMEM((ZR, D), jnp.float32),
        pltpu.VMEM_SHARED((N_NODES, D), jnp.float32),
        pltpu.SemaphoreType.DMA,
        pltpu.SemaphoreType.DMA,
        pltpu.SemaphoreType.DMA,
        pltpu.SemaphoreType.DMA,
    ],
)
def _sc_agg(x_hbm, src_hbm, dst_hbm, ew_hbm, out_hbm, *scratch):
    _sc_body(x_hbm, src_hbm, dst_hbm, ew_hbm, out_hbm, *scratch)


BM = 400


def _tc_body(p_ref, w_ref, b_ref, o_ref):
    acc = p_ref[0] + p_ref[1]
    o_ref[...] = (
        jnp.dot(acc, w_ref[...], preferred_element_type=jnp.float32)
        + b_ref[...]
    )


def _tc_finish(partials, W2, b2):
    return pl.pallas_call(
        _tc_body,
        grid=(N_NODES // BM,),
        in_specs=[
            pl.BlockSpec((NC, BM, D), lambda i: (0, i, 0)),
            pl.BlockSpec((D, D), lambda i: (0, 0)),
            pl.BlockSpec((1, D), lambda i: (0, 0)),
        ],
        out_specs=pl.BlockSpec((BM, D), lambda i: (i, 0)),
        out_shape=jax.ShapeDtypeStruct((N_NODES, D), jnp.float32),
    )(partials, W2, b2.reshape(1, D))


def kernel(x, edge_index, edge_weight, W1, b1, W2, b2):
    src = edge_index[0].astype(jnp.int32).reshape(E // K, K)
    dst = edge_index[1].astype(jnp.int32).reshape(E // K, K)
    ew = edge_weight.astype(jnp.float32).reshape(E // K, K)
    partials = _sc_agg(x, src, dst, ew)
    return _tc_finish(partials, W2, b2)


# speedup vs baseline: 1.5494x; 1.0331x over previous
import functools

import jax
import jax.numpy as jnp
from jax import lax
from jax.experimental import pallas as pl
from jax.experimental.pallas import tpu as pltpu
from jax.experimental.pallas import tpu_sc as plsc

N_NODES = 10000
D = 128
E = 320000
NC = 2
NS = 16
NW = NC * NS
K = 128
EPW = E // NW
FCPW = EPW // K
TAIL = EPW - FCPW * K
TILE_ROWS = N_NODES // NS
ZR = 25


def _scale_rows(rows_ref, b, ew_ref, eb, nrows):
    @plsc.parallel_loop(0, nrows // 16)
    def rowfn(g):
        wv = ew_ref[eb, pl.ds(16 * g, 16)]
        ws = [wv[l] for l in range(16)]
        for l in range(16):
            r = 16 * g + l
            for j in range(8):
                sl = pl.ds(16 * j, 16)
                rows_ref[b, r, sl] = rows_ref[b, r, sl] * ws[l]


def _sc_body(x_hbm, src_hbm, dst_hbm, ew_hbm, out_hbm,
             rows_v, src_v, dst_v, ew_v, trows_v, tsrc_v, tdst_v, tew_v,
             zero_v, agg_sh, g0, g1, m0, m1):
    c = lax.axis_index("c")
    s = lax.axis_index("s")
    wid = s * NC + c

    def zb(i, carry):
        for j in range(8):
            zero_v[i, pl.ds(16 * j, 16)] = jnp.zeros((16,), jnp.float32)
        return carry
    lax.fori_loop(0, ZR, zb, 0)
    base = s * TILE_ROWS
    for k in range(TILE_ROWS // ZR):
        pltpu.sync_copy(zero_v, agg_sh.at[pl.ds(base + k * ZR, ZR)])

    ebase = wid * EPW

    def meta_fetch(b, ci, sem):
        off = ebase + ci * K
        pltpu.async_copy(src_hbm.at[pl.ds(off, K)], src_v.at[b], sem)
        pltpu.async_copy(dst_hbm.at[pl.ds(off, K)], dst_v.at[b], sem)
        pltpu.async_copy(ew_hbm.at[pl.ds(off, K)], ew_v.at[b], sem)

    def meta_wait(b, ci, sem):
        off = ebase + ci * K
        pltpu.make_async_copy(
            src_hbm.at[pl.ds(off, K)], src_v.at[b], sem).wait()
        pltpu.make_async_copy(
            dst_hbm.at[pl.ds(off, K)], dst_v.at[b], sem).wait()
        pltpu.make_async_copy(
            ew_hbm.at[pl.ds(off, K)], ew_v.at[b], sem).wait()

    def gather(b, ci, gsem, msem):
        meta_wait(b, ci, msem)
        pltpu.async_copy(x_hbm.at[src_v.at[b]], rows_v.at[b], gsem)

    def process(b, ci, gsem, msem):
        pltpu.make_async_copy(
            x_hbm.at[src_v.at[b]], rows_v.at[b], gsem).wait()
        _scale_rows(rows_v, b, ew_v, b, K)
        pltpu.sync_copy(rows_v.at[b], agg_sh.at[dst_v.at[b]], add=True)
        ci_next = jnp.minimum(ci + 2, FCPW - 1)
        meta_fetch(b, ci_next, msem)

    plsc.subcore_barrier()

    toff = ebase + FCPW * K
    pltpu.sync_copy(src_hbm.at[pl.ds(toff, TAIL)], tsrc_v.at[0])
    pltpu.sync_copy(dst_hbm.at[pl.ds(toff, TAIL)], tdst_v.at[0])
    pltpu.sync_copy(ew_hbm.at[pl.ds(toff, TAIL)], tew_v.at[0])
    pltpu.async_copy(x_hbm.at[tsrc_v.at[0]], trows_v.at[0], g0).wait()
    _scale_rows(trows_v, 0, tew_v, 0, TAIL)
    pltpu.sync_copy(trows_v.at[0], agg_sh.at[tdst_v.at[0]], add=True)

    meta_fetch(0, 0, m0)
    meta_fetch(1, 1, m1)
    gather(0, 0, g0, m0)

    def pair(t, carry):
        ca = 2 * t
        gather(1, ca + 1, g1, m1)
        process(0, ca, g0, m0)
        gather(0, ca + 2, g0, m0)
        process(1, ca + 1, g1, m1)
        return carry
    lax.fori_loop(0, (FCPW - 2) // 2, pair, 0)
    gather(1, FCPW - 1, g1, m1)
    process(0, FCPW - 2, g0, m0)
    process(1, FCPW - 1, g1, m1)
    meta_wait(0, FCPW - 1, m0)
    meta_wait(1, FCPW - 1, m1)

    plsc.subcore_barrier()
    for k in range(TILE_ROWS // ZR):
        pltpu.sync_copy(agg_sh.at[pl.ds(base + k * ZR, ZR)],
                        out_hbm.at[c, pl.ds(base + k * ZR, ZR)])


@functools.partial(
    pl.kernel,
    out_type=jax.ShapeDtypeStruct((NC, N_NODES, D), jnp.float32),
    mesh=plsc.VectorSubcoreMesh(core_axis_name="c", subcore_axis_name="s"),
    compiler_params=pltpu.CompilerParams(use_tc_tiling_on_sc=False,
                                         needs_layout_passes=False),
    scratch_types=[
        pltpu.VMEM((2, K, D), jnp.float32),
        pltpu.VMEM((2, K), jnp.int32),
        pltpu.VMEM((2, K), jnp.int32),
        pltpu.VMEM((2, K), jnp.float32),
        pltpu.VMEM((1, TAIL, D), jnp.float32),
        pltpu.VMEM((1, TAIL), jnp.int32),
        pltpu.VMEM((1, TAIL), jnp.int32),
        pltpu.VMEM((1, TAIL), jnp.float32),
        pltpu.VMEM((ZR, D), jnp.float32),
        pltpu.VMEM_SHARED((N_NODES, D), jnp.float32),
        pltpu.SemaphoreType.DMA,
        pltpu.SemaphoreType.DMA,
        pltpu.SemaphoreType.DMA,
        pltpu.SemaphoreType.DMA,
    ],
)
def _sc_agg(x_hbm, src_hbm, dst_hbm, ew_hbm, out_hbm, *scratch):
    _sc_body(x_hbm, src_hbm, dst_hbm, ew_hbm, out_hbm, *scratch)


BM = 400


def _tc_body(p_ref, w_ref, b_ref, o_ref):
    acc = p_ref[0] + p_ref[1]
    o_ref[...] = (
        jnp.dot(acc, w_ref[...], preferred_element_type=jnp.float32)
        + b_ref[...]
    )


def _tc_finish(partials, W2, b2):
    return pl.pallas_call(
        _tc_body,
        grid=(N_NODES // BM,),
        in_specs=[
            pl.BlockSpec((NC, BM, D), lambda i: (0, i, 0)),
            pl.BlockSpec((D, D), lambda i: (0, 0)),
            pl.BlockSpec((1, D), lambda i: (0, 0)),
        ],
        out_specs=pl.BlockSpec((BM, D), lambda i: (i, 0)),
        out_shape=jax.ShapeDtypeStruct((N_NODES, D), jnp.float32),
    )(partials, W2, b2.reshape(1, D))


def kernel(x, edge_index, edge_weight, W1, b1, W2, b2):
    src = edge_index[0].astype(jnp.int32).reshape(E)
    dst = edge_index[1].astype(jnp.int32).reshape(E)
    ew = edge_weight.astype(jnp.float32).reshape(E)
    partials = _sc_agg(x, src, dst, ew)
    return _tc_finish(partials, W2, b2)
